# baseline (speedup 1.0000x reference)
import functools

import jax
import jax.numpy as jnp
from jax.experimental import pallas as pl

_N = 4096
_L = 200
_R = 256
_K = 8


def _sort_key(t):
    s = jax.lax.bitcast_convert_type(t, jnp.int32)
    m = jax.lax.shift_right_arithmetic(s, 31)
    return jax.lax.bitcast_convert_type(s ^ (m | jnp.int32(-2147483648)),
                                        jnp.uint32)


def _body(p_ref, t_ref, out_ref):
    i = pl.program_id(0)
    p = p_ref[...]
    t = t_ref[...]
    m = jnp.max(p, axis=1, keepdims=True)
    e = jnp.exp(p - m)
    sum_p = jnp.sum(p, axis=1)
    kappa = _sort_key(t)

    acc = jnp.zeros((_R,), jnp.float32)
    for c in range(_L // _K):
        kap_k = kappa[:, c * _K:(c + 1) * _K]
        iota_l = jax.lax.broadcasted_iota(jnp.uint32, (_K, _L), 1)
        iota_k = jax.lax.broadcasted_iota(jnp.uint32, (_K, _L), 0)
        delta = (iota_l <= iota_k + c * _K).astype(jnp.uint32)
        lhs = kappa[:, None, :] + delta[None]
        cmp = lhs > kap_k[:, :, None]
        tk = jnp.sum(jnp.where(cmp, e[:, None, :], 0.0), axis=2)
        acc = acc + jnp.sum(jnp.log(tk), axis=1)

    row_loss = -sum_p + acc + jnp.float32(_L) * m[:, 0]
    partial = jnp.sum(row_loss) * jnp.float32(1.0 / _N)

    @pl.when(i == 0)
    def _():
        out_ref[...] = jnp.zeros_like(out_ref)

    out_ref[...] += jnp.full((1, 1), partial, jnp.float32)


@jax.jit
def kernel(y_pred, y_true):
    out = pl.pallas_call(
        _body,
        grid=(_N // _R,),
        in_specs=[
            pl.BlockSpec((_R, _L), lambda i: (i, 0)),
            pl.BlockSpec((_R, _L), lambda i: (i, 0)),
        ],
        out_specs=pl.BlockSpec((1, 1), lambda i: (0, 0)),
        out_shape=jax.ShapeDtypeStruct((1, 1), jnp.float32),
    )(y_pred, y_true)
    return out[0, 0]


# speedup vs baseline: 2.6432x; 2.6432x over previous
import functools

import jax
import jax.numpy as jnp
from jax.experimental import pallas as pl
from jax.experimental.pallas import tpu as pltpu

_N = 4096
_L = 200
_C = 512


def _sort_key(t):
    s = jax.lax.bitcast_convert_type(t, jnp.int32)
    m = jax.lax.shift_right_arithmetic(s, 31)
    return jax.lax.bitcast_convert_type(s ^ (m | jnp.int32(-2147483648)),
                                        jnp.uint32)


def _body(pt_ref, tt_ref, out_ref, kap_ref, lhs_ref):
    i = pl.program_id(0)
    p = pt_ref[...]
    t = tt_ref[...]
    m = jnp.max(p, axis=0, keepdims=True)
    e = jnp.exp(p - m)
    sum_p = jnp.sum(p, axis=0, keepdims=True)
    kappa = _sort_key(t)

    iota_l = jax.lax.broadcasted_iota(jnp.uint32, (_L, _C), 0)
    kap_ref[: _L, :] = kappa
    lhs_ref[: _L, :] = kappa + (iota_l == 0).astype(jnp.uint32)

    def step(k, acc):
        kap_k = kap_ref[pl.ds(k, 1), :]
        cmp = lhs_ref[: _L, :] > kap_k
        tk = jnp.sum(jnp.where(cmp, e, 0.0), axis=0, keepdims=True)
        acc = acc + jnp.log(tk)
        lhs_ref[pl.ds(k + 1, 1), :] += jnp.uint32(1)
        return acc

    acc = jax.lax.fori_loop(
        0, _L, step, jnp.zeros((1, _C), jnp.float32))

    col_loss = -sum_p + acc + jnp.float32(_L) * m
    partial = jnp.sum(col_loss) * jnp.float32(1.0 / _N)

    @pl.when(i == 0)
    def _():
        out_ref[...] = jnp.zeros_like(out_ref)

    out_ref[...] += jnp.full((1, 1), partial, jnp.float32)


@jax.jit
def kernel(y_pred, y_true):
    pt = y_pred.T
    tt = y_true.T
    out = pl.pallas_call(
        _body,
        grid=(_N // _C,),
        in_specs=[
            pl.BlockSpec((_L, _C), lambda i: (0, i)),
            pl.BlockSpec((_L, _C), lambda i: (0, i)),
        ],
        out_specs=pl.BlockSpec((1, 1), lambda i: (0, 0)),
        out_shape=jax.ShapeDtypeStruct((1, 1), jnp.float32),
        scratch_shapes=[
            pltpu.VMEM((_L + 8, _C), jnp.uint32),
            pltpu.VMEM((_L + 8, _C), jnp.uint32),
        ],
    )(pt, tt)
    return out[0, 0]


# speedup vs baseline: 2.6789x; 1.0135x over previous
import functools

import jax
import jax.numpy as jnp
from jax.experimental import pallas as pl
from jax.experimental.pallas import tpu as pltpu

_N = 4096
_L = 200
_C = 1024


def _sort_key(t):
    s = jax.lax.bitcast_convert_type(t, jnp.int32)
    m = jax.lax.shift_right_arithmetic(s, 31)
    return jax.lax.bitcast_convert_type(s ^ (m | jnp.int32(-2147483648)),
                                        jnp.uint32)


def _body(pt_ref, tt_ref, out_ref, kap_ref, lhs_ref, tmat_ref):
    i = pl.program_id(0)
    p = pt_ref[...]
    t = tt_ref[...]
    m = jnp.max(p, axis=0, keepdims=True)
    e = jnp.exp(p - m)
    sum_p = jnp.sum(p, axis=0, keepdims=True)
    kappa = _sort_key(t)

    iota_l = jax.lax.broadcasted_iota(jnp.uint32, (_L, _C), 0)
    kap_ref[: _L, :] = kappa
    lhs_ref[: _L, :] = kappa + (iota_l == 0).astype(jnp.uint32)

    def step(k, carry):
        kap_k = kap_ref[pl.ds(k, 1), :]
        cmp = lhs_ref[: _L, :] > kap_k
        tk = jnp.sum(jnp.where(cmp, e, 0.0), axis=0, keepdims=True)
        tmat_ref[pl.ds(k, 1), :] = tk
        lhs_ref[pl.ds(k + 1, 1), :] += jnp.uint32(1)
        return carry

    jax.lax.fori_loop(0, _L, step, 0)

    acc = jnp.sum(jnp.log(tmat_ref[: _L, :]), axis=0, keepdims=True)
    col_loss = -sum_p + acc + jnp.float32(_L) * m
    partial = jnp.sum(col_loss) * jnp.float32(1.0 / _N)

    @pl.when(i == 0)
    def _():
        out_ref[...] = jnp.zeros_like(out_ref)

    out_ref[...] += jnp.full((1, 1), partial, jnp.float32)


@jax.jit
def kernel(y_pred, y_true):
    pt = y_pred.T
    tt = y_true.T
    out = pl.pallas_call(
        _body,
        grid=(_N // _C,),
        in_specs=[
            pl.BlockSpec((_L, _C), lambda i: (0, i)),
            pl.BlockSpec((_L, _C), lambda i: (0, i)),
        ],
        out_specs=pl.BlockSpec((1, 1), lambda i: (0, 0)),
        out_shape=jax.ShapeDtypeStruct((1, 1), jnp.float32),
        scratch_shapes=[
            pltpu.VMEM((_L + 8, _C), jnp.uint32),
            pltpu.VMEM((_L + 8, _C), jnp.uint32),
            pltpu.VMEM((_L + 8, _C), jnp.float32),
        ],
    )(pt, tt)
    return out[0, 0]


# speedup vs baseline: 4.0848x; 1.5248x over previous
import functools

import jax
import jax.numpy as jnp
from jax.experimental import pallas as pl
from jax.experimental.pallas import tpu as pltpu

_N = 4096
_L = 200
_C = 256
_G = 8
_NT = _L // _G


def _sort_key(t):
    s = jax.lax.bitcast_convert_type(t, jnp.int32)
    m = jax.lax.shift_right_arithmetic(s, 31)
    return jax.lax.bitcast_convert_type(s ^ (m | jnp.int32(-2147483648)),
                                        jnp.uint32)


def _body(pt_ref, tt_ref, out_ref, kap_ref, e_ref, tmat_ref):
    i = pl.program_id(0)
    p = pt_ref[...]
    t = tt_ref[...]
    m = jnp.max(p, axis=0, keepdims=True)
    e_ref[: _L, :] = jnp.exp(p - m)
    sum_p = jnp.sum(p, axis=0, keepdims=True)
    kap_ref[: _L, :] = _sort_key(t)

    iota8 = jax.lax.broadcasted_iota(jnp.uint32, (_G, _C), 0)

    def group(g, carry):
        base = g * _G
        kq = kap_ref[pl.ds(base, _G), :]
        eq8 = e_ref[pl.ds(base, _G), :]

        kqb = [jnp.broadcast_to(kq[j:j + 1, :], (_G, _C)) for j in range(_G)]

        accs = [jnp.zeros((_G, _C), jnp.float32) for _ in range(_G)]
        for tile in range(_NT):
            lhs = kap_ref[tile * _G:(tile + 1) * _G, :]
            et = e_ref[tile * _G:(tile + 1) * _G, :]
            for j in range(_G):
                accs[j] = accs[j] + jnp.where(lhs > kqb[j], et, 0.0)

        tks = []
        for j in range(_G):
            corr = jnp.where((kq == kqb[j]) & (iota8 <= j), eq8, 0.0)
            tks.append(jnp.sum(accs[j] + corr, axis=0, keepdims=True))
        tmat_ref[pl.ds(base, _G), :] = jnp.concatenate(tks, axis=0)

        kap_ref[pl.ds(base, _G), :] = kq + jnp.uint32(1)
        return carry

    jax.lax.fori_loop(0, _NT, group, 0)

    acc = jnp.sum(jnp.log(tmat_ref[: _L, :]), axis=0, keepdims=True)
    col_loss = -sum_p + acc + jnp.float32(_L) * m
    partial = jnp.sum(col_loss) * jnp.float32(1.0 / _N)

    @pl.when(i == 0)
    def _():
        out_ref[...] = jnp.zeros_like(out_ref)

    out_ref[...] += jnp.full((1, 1), partial, jnp.float32)


@jax.jit
def kernel(y_pred, y_true):
    pt = y_pred.T
    tt = y_true.T
    out = pl.pallas_call(
        _body,
        grid=(_N // _C,),
        in_specs=[
            pl.BlockSpec((_L, _C), lambda i: (0, i)),
            pl.BlockSpec((_L, _C), lambda i: (0, i)),
        ],
        out_specs=pl.BlockSpec((1, 1), lambda i: (0, 0)),
        out_shape=jax.ShapeDtypeStruct((1, 1), jnp.float32),
        scratch_shapes=[
            pltpu.VMEM((_L, _C), jnp.uint32),
            pltpu.VMEM((_L, _C), jnp.float32),
            pltpu.VMEM((_L, _C), jnp.float32),
        ],
    )(pt, tt)
    return out[0, 0]
